# baseline (speedup 1.0000x reference)
import jax
import jax.numpy as jnp
from jax.experimental import pallas as pl
from jax.experimental.pallas import tpu as pltpu

_B, _C, _H, _E = 256, 1024, 32, 32
_D = _H + _E
_ET = 2
_BT = 32
_G = 16


def _sig(x):
    return 0.5 * jnp.tanh(0.5 * x) + 0.5


def _prologue_kernel(x_emb_ref, se_w1_ref, se_w2_ref, emb_c_ref,
                     wq_ref, qtf_ref, xt_ref, qt_kc_ref, graphs_ref,
                     aux_ref, mf_ref, adj0_ref, adj1_ref, res_ref):
    x = x_emb_ref[...]
    s_col = jnp.mean(x, axis=1, keepdims=True)
    s_row = jnp.transpose(s_col)
    h1 = jnp.maximum(jnp.dot(s_row, se_w1_ref[...],
                             preferred_element_type=jnp.float32), 0.0)
    scale_row = _sig(jnp.dot(h1, se_w2_ref[...],
                             preferred_element_type=jnp.float32))
    sc_x_emb = x * jnp.transpose(scale_row)

    e0 = jnp.transpose(emb_c_ref[0:1, :])
    e1 = jnp.transpose(emb_c_ref[1:2, :])
    ecols = jnp.concatenate([e0, e1 - e0, e1], axis=1)
    aux_ref[...] = jnp.dot(wq_ref[...], ecols,
                           preferred_element_type=jnp.float32)

    lane_iota = jax.lax.broadcasted_iota(
        jnp.int32, (_B, _C), 1).astype(jnp.float32)
    onehot = (lane_iota == qtf_ref[...]).astype(jnp.float32)
    mf = jnp.dot(onehot, qt_kc_ref[...],
                 preferred_element_type=jnp.float32)
    mf_ref[...] = mf

    denom = jnp.maximum(jnp.sum(mf, axis=1, keepdims=True), 1.0)
    mfn = mf * (1.0 / denom)
    adj0_ref[...] = jnp.dot(mfn, graphs_ref[0],
                            preferred_element_type=jnp.float32)
    adj1_ref[...] = jnp.dot(mfn, graphs_ref[1],
                            preferred_element_type=jnp.float32)

    res_ref[...] = jnp.dot(mf * xt_ref[...], sc_x_emb,
                           preferred_element_type=jnp.float32)


def _gkt_kernel(qt_s,
                ht_ref, mf_ref, adj0_ref, adj1_ref, res_ref,
                aux_ref, whtcat_ref, bhh_ref,
                pa_w_ref, fsw2aug_ref, w2aug_ref,
                eawaug_ref, wihaug_ref, eaw_row_ref, wpaug_ref,
                out_ref):
    i = pl.program_id(0)
    base = i * _BT

    qe0 = aux_ref[:, 0:1]
    qd = aux_ref[:, 1:2]
    qe1 = aux_ref[:, 2:3]
    lane_iota = jax.lax.broadcasted_iota(jnp.int32, (1, _C), 1)
    ones_c = jnp.ones((1, _C), jnp.float32)
    ones_bt = jnp.ones((1, _BT), jnp.float32)
    w_row = eaw_row_ref[...]

    ceq_rows = []
    htq_cols = []
    for j in range(_BT):
        q = qt_s[base + j]
        ceq = (lane_iota == q).astype(jnp.float32)
        ceq_rows.append(ceq)
        htq_cols.append(jnp.dot(ht_ref[j], jnp.transpose(ceq),
                                preferred_element_type=jnp.float32))
    scols_aug = jnp.concatenate(
        [jnp.concatenate(htq_cols, axis=1),
         jnp.transpose(res_ref[...]), ones_bt], axis=0)

    pa = jnp.dot(pa_w_ref[...], scols_aug,
                 preferred_element_type=jnp.float32)
    f1 = jnp.maximum(pa[:_H], 0.0)
    self_feat_all = jnp.dot(fsw2aug_ref[...],
                            jnp.concatenate([f1, ones_bt], axis=0),
                            preferred_element_type=jnp.float32)
    colones_fn = pa[_H:3 * _H] + qe0
    colceq_fn = pa[3 * _H:] - qe1
    cols_gru = jnp.concatenate(
        [bhh_ref[...], jnp.zeros((3 * _H, 2), jnp.float32)],
        axis=1)

    ones_gc = jnp.ones((1, _G * _C), jnp.float32)
    w_all = jnp.concatenate([w_row] * _G, axis=1)
    blk_iota = jax.lax.broadcasted_iota(
        jnp.int32, (_G, _G * _C), 1) // _C
    sub_iota = jax.lax.broadcasted_iota(
        jnp.int32, (_G, _G * _C), 0)
    blk_sel = (sub_iota == blk_iota).astype(jnp.float32)

    out_groups = []
    for g in range(_BT // _G):
        js = range(g * _G, (g + 1) * _G)
        z_parts, gh_parts = [], []
        for j in js:
            ceq_row = ceq_rows[j]
            mf_row = mf_ref[j:j + 1, :]
            adj0_row = adj0_ref[j:j + 1, :]
            adj1_row = adj1_ref[j:j + 1, :]

            cols_fn = jnp.concatenate(
                [colones_fn[:, j:j + 1], qd, colceq_fn[:, j:j + 1]],
                axis=1)
            w_aug = jnp.concatenate(
                [whtcat_ref[...],
                 jnp.concatenate([cols_fn, cols_gru], axis=0)], axis=1)
            aug_in = jnp.concatenate(
                [ht_ref[j], ones_c, mf_row, ceq_row], axis=0)
            big = jnp.dot(w_aug, aug_in,
                          preferred_element_type=jnp.float32)
            h1 = jnp.maximum(big[:2 * _H], 0.0)
            gh_parts.append(big[2 * _H:])

            z_parts.append(jnp.concatenate(
                [h1[:_H] * adj0_row, h1[_H:] * adj1_row,
                 adj0_row, adj1_row], axis=0))

        z_all = jnp.concatenate(z_parts, axis=1)
        gh_all = jnp.concatenate(gh_parts, axis=1)
        ceq_all = jnp.concatenate(
            [ceq_rows[j] for j in js], axis=1)
        htT_all = jnp.concatenate(
            [ht_ref[j] for j in js], axis=1)

        nf_all = jnp.dot(w2aug_ref[...], z_all,
                         preferred_element_type=jnp.float32)

        t_g = blk_sel * ceq_all
        outer_all = jnp.dot(self_feat_all[:, g * _G:(g + 1) * _G], t_g,
                            preferred_element_type=jnp.float32)
        m_next = nf_all * (1.0 - ceq_all) + outer_all

        ea = jnp.dot(eawaug_ref[...],
                     jnp.concatenate([m_next, ones_gc], axis=0),
                     preferred_element_type=jnp.float32)
        eg = _sig(ea[:_H])
        ag = jnp.tanh(ea[_H:])
        m2 = m_next - (w_all * eg) * m_next + w_all * ag

        gi = jnp.dot(wihaug_ref[...],
                     jnp.concatenate([m2, ones_gc], axis=0),
                     preferred_element_type=jnp.float32)
        r = _sig(gi[:_H] + gh_all[:_H])
        zg = _sig(gi[_H:2 * _H] + gh_all[_H:2 * _H])
        n = jnp.tanh(gi[2 * _H:] + r * gh_all[2 * _H:])
        h_next = n + zg * (htT_all - n)

        out_groups.append(_sig(jnp.dot(
            wpaug_ref[...],
            jnp.concatenate([h_next, ones_gc], axis=0),
            preferred_element_type=jnp.float32)))

    out_ref[0] = jnp.concatenate(out_groups, axis=1)


def kernel(xt, qt, ht, qt_kc, emb_x_table, emb_c_table, se_w1, se_w2,
           fs_w1, fs_b1, fs_w2, fs_b2, fn_w1, fn_b1, fn_w2, fn_b2,
           ea_w, ea_we, ea_be, ea_wa, ea_ba,
           gru_wih, gru_bih, gru_whh, gru_bhh, wp, bp, graphs):
    f32 = jnp.float32
    x_emb = emb_x_table[:_C]

    wh_T = jnp.concatenate([fn_w1[0, _D:_D + _H].T,
                            fn_w1[1, _D:_D + _H].T], axis=0)
    wq_T = jnp.concatenate([fn_w1[0, _D + _H:].T,
                            fn_w1[1, _D + _H:].T], axis=0)
    wself_T = jnp.concatenate([fn_w1[0, :_D].T,
                               fn_w1[1, :_D].T], axis=0)
    b1cat = jnp.concatenate([fn_b1[0], fn_b1[1]]).reshape(2 * _H, 1)
    w2cat_T = jnp.concatenate([fn_w2[0].T, fn_w2[1].T], axis=1)
    b2_T = jnp.stack([fn_b2[0], fn_b2[1]], axis=1)
    eacat_T = jnp.concatenate([ea_we.T, ea_wa.T], axis=0)
    whtcat = jnp.concatenate([wh_T, gru_whh.T], axis=0)

    pa_w = jnp.concatenate([
        jnp.concatenate([fs_w1.T, fs_b1.reshape(_H, 1)], axis=1),
        jnp.concatenate([wself_T, b1cat], axis=1),
        jnp.concatenate([jnp.zeros((2 * _H, _H), f32), wq_T,
                         jnp.zeros((2 * _H, 1), f32)], axis=1),
    ], axis=0)
    fsw2_aug = jnp.concatenate([fs_w2.T, fs_b2.reshape(_H, 1)], axis=1)
    w2aug = jnp.concatenate([w2cat_T, b2_T], axis=1)
    ea_w_aug = jnp.concatenate(
        [eacat_T,
         jnp.concatenate([ea_be, ea_ba]).reshape(2 * _H, 1)], axis=1)
    wih_aug = jnp.concatenate([gru_wih.T, gru_bih.reshape(3 * _H, 1)],
                              axis=1)
    wp_aug = jnp.concatenate([wp.reshape(1, _H), bp.reshape(1, 1)],
                             axis=1)

    aux, mf, adj0, adj1, res = pl.pallas_call(
        _prologue_kernel,
        out_shape=(jax.ShapeDtypeStruct((2 * _H, 3), f32),
                   jax.ShapeDtypeStruct((_B, _C), f32),
                   jax.ShapeDtypeStruct((_B, _C), f32),
                   jax.ShapeDtypeStruct((_B, _C), f32),
                   jax.ShapeDtypeStruct((_B, _E), f32)),
    )(x_emb, se_w1, se_w2, emb_c_table, wq_T,
      qt.astype(f32).reshape(_B, 1), xt.reshape(_B, 1),
      qt_kc[:_C], graphs)

    operands = (
        jnp.transpose(ht, (0, 2, 1)), mf, adj0, adj1, res,
        aux, whtcat, gru_bhh.reshape(3 * _H, 1),
        pa_w, fsw2_aug, w2aug,
        ea_w_aug, wih_aug, ea_w.reshape(1, _C), wp_aug,
    )

    def full(a):
        nd = a.ndim
        return pl.BlockSpec(a.shape, lambda i, q, _n=nd: (0,) * _n)

    in_specs = [
        pl.BlockSpec((_BT, _H, _C), lambda i, q: (i, 0, 0)),
        pl.BlockSpec((_BT, _C), lambda i, q: (i, 0)),
        pl.BlockSpec((_BT, _C), lambda i, q: (i, 0)),
        pl.BlockSpec((_BT, _C), lambda i, q: (i, 0)),
        pl.BlockSpec((_BT, _E), lambda i, q: (i, 0)),
    ] + [full(a) for a in operands[5:]]

    grid_spec = pltpu.PrefetchScalarGridSpec(
        num_scalar_prefetch=1,
        grid=(_B // _BT,),
        in_specs=in_specs,
        out_specs=pl.BlockSpec((1, 1, _BT * _C), lambda i, q: (i, 0, 0)),
    )

    yt = pl.pallas_call(
        _gkt_kernel,
        grid_spec=grid_spec,
        out_shape=jax.ShapeDtypeStruct((_B // _BT, 1, _BT * _C), f32),
    )(qt, *operands)
    return yt.reshape(_B, _C)


# speedup vs baseline: 1.0081x; 1.0081x over previous
import jax
import jax.numpy as jnp
from jax.experimental import pallas as pl
from jax.experimental.pallas import tpu as pltpu

_B, _C, _H, _E = 256, 1024, 32, 32
_D = _H + _E
_ET = 2
_BT = 64
_G = 8


def _sig(x):
    return 0.5 * jnp.tanh(0.5 * x) + 0.5


def _prologue_kernel(x_emb_ref, se_w1_ref, se_w2_ref, emb_c_ref,
                     wq_ref, qtf_ref, xt_ref, qt_kc_ref, graphs_ref,
                     aux_ref, mf_ref, adj0_ref, adj1_ref, res_ref):
    x = x_emb_ref[...]
    s_col = jnp.mean(x, axis=1, keepdims=True)
    s_row = jnp.transpose(s_col)
    h1 = jnp.maximum(jnp.dot(s_row, se_w1_ref[...],
                             preferred_element_type=jnp.float32), 0.0)
    scale_row = _sig(jnp.dot(h1, se_w2_ref[...],
                             preferred_element_type=jnp.float32))
    sc_x_emb = x * jnp.transpose(scale_row)

    e0 = jnp.transpose(emb_c_ref[0:1, :])
    e1 = jnp.transpose(emb_c_ref[1:2, :])
    ecols = jnp.concatenate([e0, e1 - e0, e1], axis=1)
    aux_ref[...] = jnp.dot(wq_ref[...], ecols,
                           preferred_element_type=jnp.float32)

    lane_iota = jax.lax.broadcasted_iota(
        jnp.int32, (_B, _C), 1).astype(jnp.float32)
    onehot = (lane_iota == qtf_ref[...]).astype(jnp.float32)
    mf = jnp.dot(onehot, qt_kc_ref[...],
                 preferred_element_type=jnp.float32)
    mf_ref[...] = mf

    denom = jnp.maximum(jnp.sum(mf, axis=1, keepdims=True), 1.0)
    mfn = mf * (1.0 / denom)
    adj0_ref[...] = jnp.dot(mfn, graphs_ref[0],
                            preferred_element_type=jnp.float32)
    adj1_ref[...] = jnp.dot(mfn, graphs_ref[1],
                            preferred_element_type=jnp.float32)

    res_ref[...] = jnp.dot(mf * xt_ref[...], sc_x_emb,
                           preferred_element_type=jnp.float32)


def _gkt_kernel(qt_s,
                ht_ref, mf_ref, adj0_ref, adj1_ref, res_ref,
                aux_ref, whtcat_ref, bhh_ref,
                pa_w_ref, fsw2aug_ref, w2aug_ref,
                eawaug_ref, wihaug_ref, eaw_row_ref, wpaug_ref,
                out_ref):
    i = pl.program_id(0)
    base = i * _BT

    qe0 = aux_ref[:, 0:1]
    qd = aux_ref[:, 1:2]
    qe1 = aux_ref[:, 2:3]
    lane_iota = jax.lax.broadcasted_iota(jnp.int32, (1, _C), 1)
    ones_c = jnp.ones((1, _C), jnp.float32)
    ones_bt = jnp.ones((1, _BT), jnp.float32)
    w_row = eaw_row_ref[...]

    ceq_rows = []
    htq_cols = []
    for j in range(_BT):
        q = qt_s[base + j]
        ceq = (lane_iota == q).astype(jnp.float32)
        ceq_rows.append(ceq)
        htq_cols.append(jnp.dot(ht_ref[j], jnp.transpose(ceq),
                                preferred_element_type=jnp.float32))
    scols_aug = jnp.concatenate(
        [jnp.concatenate(htq_cols, axis=1),
         jnp.transpose(res_ref[...]), ones_bt], axis=0)

    pa = jnp.dot(pa_w_ref[...], scols_aug,
                 preferred_element_type=jnp.float32)
    f1 = jnp.maximum(pa[:_H], 0.0)
    self_feat_all = jnp.dot(fsw2aug_ref[...],
                            jnp.concatenate([f1, ones_bt], axis=0),
                            preferred_element_type=jnp.float32)
    colones_fn = pa[_H:3 * _H] + qe0
    colceq_fn = pa[3 * _H:] - qe1
    cols_gru = jnp.concatenate(
        [bhh_ref[...], jnp.zeros((3 * _H, 2), jnp.float32)],
        axis=1)

    ones_gc = jnp.ones((1, _G * _C), jnp.float32)
    w_all = jnp.concatenate([w_row] * _G, axis=1)
    blk_iota = jax.lax.broadcasted_iota(
        jnp.int32, (_G, _G * _C), 1) // _C
    sub_iota = jax.lax.broadcasted_iota(
        jnp.int32, (_G, _G * _C), 0)
    blk_sel = (sub_iota == blk_iota).astype(jnp.float32)

    out_groups = []
    for g in range(_BT // _G):
        js = range(g * _G, (g + 1) * _G)
        z_parts, gh_parts = [], []
        for j in js:
            ceq_row = ceq_rows[j]
            mf_row = mf_ref[j:j + 1, :]
            adj0_row = adj0_ref[j:j + 1, :]
            adj1_row = adj1_ref[j:j + 1, :]

            cols_fn = jnp.concatenate(
                [colones_fn[:, j:j + 1], qd, colceq_fn[:, j:j + 1]],
                axis=1)
            w_aug = jnp.concatenate(
                [whtcat_ref[...],
                 jnp.concatenate([cols_fn, cols_gru], axis=0)], axis=1)
            aug_in = jnp.concatenate(
                [ht_ref[j], ones_c, mf_row, ceq_row], axis=0)
            big = jnp.dot(w_aug, aug_in,
                          preferred_element_type=jnp.float32)
            h1 = jnp.maximum(big[:2 * _H], 0.0)
            gh_parts.append(big[2 * _H:])

            z_parts.append(jnp.concatenate(
                [h1[:_H] * adj0_row, h1[_H:] * adj1_row,
                 adj0_row, adj1_row], axis=0))

        z_all = jnp.concatenate(z_parts, axis=1)
        gh_all = jnp.concatenate(gh_parts, axis=1)
        ceq_all = jnp.concatenate(
            [ceq_rows[j] for j in js], axis=1)
        htT_all = jnp.concatenate(
            [ht_ref[j] for j in js], axis=1)

        nf_all = jnp.dot(w2aug_ref[...], z_all,
                         preferred_element_type=jnp.float32)

        t_g = blk_sel * ceq_all
        outer_all = jnp.dot(self_feat_all[:, g * _G:(g + 1) * _G], t_g,
                            preferred_element_type=jnp.float32)
        m_next = nf_all * (1.0 - ceq_all) + outer_all

        ea = jnp.dot(eawaug_ref[...],
                     jnp.concatenate([m_next, ones_gc], axis=0),
                     preferred_element_type=jnp.float32)
        eg = _sig(ea[:_H])
        ag = jnp.tanh(ea[_H:])
        m2 = m_next - (w_all * eg) * m_next + w_all * ag

        gi = jnp.dot(wihaug_ref[...],
                     jnp.concatenate([m2, ones_gc], axis=0),
                     preferred_element_type=jnp.float32)
        r = _sig(gi[:_H] + gh_all[:_H])
        zg = _sig(gi[_H:2 * _H] + gh_all[_H:2 * _H])
        n = jnp.tanh(gi[2 * _H:] + r * gh_all[2 * _H:])
        h_next = n + zg * (htT_all - n)

        out_groups.append(_sig(jnp.dot(
            wpaug_ref[...],
            jnp.concatenate([h_next, ones_gc], axis=0),
            preferred_element_type=jnp.float32)))

    out_ref[0] = jnp.concatenate(out_groups, axis=1)


def kernel(xt, qt, ht, qt_kc, emb_x_table, emb_c_table, se_w1, se_w2,
           fs_w1, fs_b1, fs_w2, fs_b2, fn_w1, fn_b1, fn_w2, fn_b2,
           ea_w, ea_we, ea_be, ea_wa, ea_ba,
           gru_wih, gru_bih, gru_whh, gru_bhh, wp, bp, graphs):
    f32 = jnp.float32
    x_emb = emb_x_table[:_C]

    wh_T = jnp.concatenate([fn_w1[0, _D:_D + _H].T,
                            fn_w1[1, _D:_D + _H].T], axis=0)
    wq_T = jnp.concatenate([fn_w1[0, _D + _H:].T,
                            fn_w1[1, _D + _H:].T], axis=0)
    wself_T = jnp.concatenate([fn_w1[0, :_D].T,
                               fn_w1[1, :_D].T], axis=0)
    b1cat = jnp.concatenate([fn_b1[0], fn_b1[1]]).reshape(2 * _H, 1)
    w2cat_T = jnp.concatenate([fn_w2[0].T, fn_w2[1].T], axis=1)
    b2_T = jnp.stack([fn_b2[0], fn_b2[1]], axis=1)
    eacat_T = jnp.concatenate([ea_we.T, ea_wa.T], axis=0)
    whtcat = jnp.concatenate([wh_T, gru_whh.T], axis=0)

    pa_w = jnp.concatenate([
        jnp.concatenate([fs_w1.T, fs_b1.reshape(_H, 1)], axis=1),
        jnp.concatenate([wself_T, b1cat], axis=1),
        jnp.concatenate([jnp.zeros((2 * _H, _H), f32), wq_T,
                         jnp.zeros((2 * _H, 1), f32)], axis=1),
    ], axis=0)
    fsw2_aug = jnp.concatenate([fs_w2.T, fs_b2.reshape(_H, 1)], axis=1)
    w2aug = jnp.concatenate([w2cat_T, b2_T], axis=1)
    ea_w_aug = jnp.concatenate(
        [eacat_T,
         jnp.concatenate([ea_be, ea_ba]).reshape(2 * _H, 1)], axis=1)
    wih_aug = jnp.concatenate([gru_wih.T, gru_bih.reshape(3 * _H, 1)],
                              axis=1)
    wp_aug = jnp.concatenate([wp.reshape(1, _H), bp.reshape(1, 1)],
                             axis=1)

    aux, mf, adj0, adj1, res = pl.pallas_call(
        _prologue_kernel,
        out_shape=(jax.ShapeDtypeStruct((2 * _H, 3), f32),
                   jax.ShapeDtypeStruct((_B, _C), f32),
                   jax.ShapeDtypeStruct((_B, _C), f32),
                   jax.ShapeDtypeStruct((_B, _C), f32),
                   jax.ShapeDtypeStruct((_B, _E), f32)),
    )(x_emb, se_w1, se_w2, emb_c_table, wq_T,
      qt.astype(f32).reshape(_B, 1), xt.reshape(_B, 1),
      qt_kc[:_C], graphs)

    operands = (
        jnp.transpose(ht, (0, 2, 1)), mf, adj0, adj1, res,
        aux, whtcat, gru_bhh.reshape(3 * _H, 1),
        pa_w, fsw2_aug, w2aug,
        ea_w_aug, wih_aug, ea_w.reshape(1, _C), wp_aug,
    )

    def full(a):
        nd = a.ndim
        return pl.BlockSpec(a.shape, lambda i, q, _n=nd: (0,) * _n)

    in_specs = [
        pl.BlockSpec((_BT, _H, _C), lambda i, q: (i, 0, 0)),
        pl.BlockSpec((_BT, _C), lambda i, q: (i, 0)),
        pl.BlockSpec((_BT, _C), lambda i, q: (i, 0)),
        pl.BlockSpec((_BT, _C), lambda i, q: (i, 0)),
        pl.BlockSpec((_BT, _E), lambda i, q: (i, 0)),
    ] + [full(a) for a in operands[5:]]

    grid_spec = pltpu.PrefetchScalarGridSpec(
        num_scalar_prefetch=1,
        grid=(_B // _BT,),
        in_specs=in_specs,
        out_specs=pl.BlockSpec((1, 1, _BT * _C), lambda i, q: (i, 0, 0)),
    )

    yt = pl.pallas_call(
        _gkt_kernel,
        grid_spec=grid_spec,
        out_shape=jax.ShapeDtypeStruct((_B // _BT, 1, _BT * _C), f32),
    )(qt, *operands)
    return yt.reshape(_B, _C)


# speedup vs baseline: 1.0156x; 1.0074x over previous
import jax
import jax.numpy as jnp
from jax.experimental import pallas as pl
from jax.experimental.pallas import tpu as pltpu

_B, _C, _H, _E = 256, 1024, 32, 32
_D = _H + _E
_ET = 2
_BT = 32
_G = 8


def _sig(x):
    return 0.5 * jnp.tanh(0.5 * x) + 0.5


def _sigp(x):
    return 0.5 * jnp.tanh(x) + 0.5


def _prologue_kernel(x_emb_ref, se_w1_ref, se_w2_ref, emb_c_ref,
                     wq_ref, qtf_ref, xt_ref, qt_kc_ref, graphs_ref,
                     aux_ref, mf_ref, adj0_ref, adj1_ref, res_ref):
    x = x_emb_ref[...]
    s_col = jnp.mean(x, axis=1, keepdims=True)
    s_row = jnp.transpose(s_col)
    h1 = jnp.maximum(jnp.dot(s_row, se_w1_ref[...],
                             preferred_element_type=jnp.float32), 0.0)
    scale_row = _sig(jnp.dot(h1, se_w2_ref[...],
                             preferred_element_type=jnp.float32))
    sc_x_emb = x * jnp.transpose(scale_row)

    e0 = jnp.transpose(emb_c_ref[0:1, :])
    e1 = jnp.transpose(emb_c_ref[1:2, :])
    ecols = jnp.concatenate([e0, e1 - e0, e1], axis=1)
    aux_ref[...] = jnp.dot(wq_ref[...], ecols,
                           preferred_element_type=jnp.float32)

    lane_iota = jax.lax.broadcasted_iota(
        jnp.int32, (_B, _C), 1).astype(jnp.float32)
    onehot = (lane_iota == qtf_ref[...]).astype(jnp.float32)
    mf = jnp.dot(onehot, qt_kc_ref[...],
                 preferred_element_type=jnp.float32)
    mf_ref[...] = mf

    denom = jnp.maximum(jnp.sum(mf, axis=1, keepdims=True), 1.0)
    mfn = mf * (1.0 / denom)
    adj0_ref[...] = jnp.dot(mfn, graphs_ref[0],
                            preferred_element_type=jnp.float32)
    adj1_ref[...] = jnp.dot(mfn, graphs_ref[1],
                            preferred_element_type=jnp.float32)

    res_ref[...] = jnp.dot(mf * xt_ref[...], sc_x_emb,
                           preferred_element_type=jnp.float32)


def _gkt_kernel(qt_s,
                ht_ref, mf_ref, adj0_ref, adj1_ref, res_ref,
                aux_ref, whtcat_ref, bhh_ref,
                pa_w_ref, fsw2aug_ref, w2aug_ref,
                eawaug_ref, wihaug_ref, eaw_row_ref, wpaug_ref,
                out_ref):
    i = pl.program_id(0)
    base = i * _BT

    qe0 = aux_ref[:, 0:1]
    qd = aux_ref[:, 1:2]
    qe1 = aux_ref[:, 2:3]
    lane_iota = jax.lax.broadcasted_iota(jnp.int32, (1, _C), 1)
    ones_c = jnp.ones((1, _C), jnp.float32)
    ones_bt = jnp.ones((1, _BT), jnp.float32)
    w_row = eaw_row_ref[...]

    ceq_rows = []
    htq_cols = []
    for j in range(_BT):
        q = qt_s[base + j]
        ceq = (lane_iota == q).astype(jnp.float32)
        ceq_rows.append(ceq)
        htq_cols.append(jnp.dot(ht_ref[j], jnp.transpose(ceq),
                                preferred_element_type=jnp.float32))
    scols_aug = jnp.concatenate(
        [jnp.concatenate(htq_cols, axis=1),
         jnp.transpose(res_ref[...]), ones_bt], axis=0)

    pa = jnp.dot(pa_w_ref[...], scols_aug,
                 preferred_element_type=jnp.float32)
    f1 = jnp.maximum(pa[:_H], 0.0)
    self_feat_all = jnp.dot(fsw2aug_ref[...],
                            jnp.concatenate([f1, ones_bt], axis=0),
                            preferred_element_type=jnp.float32)
    colones_fn = pa[_H:3 * _H] + qe0
    colceq_fn = pa[3 * _H:] - qe1
    cols_gru = jnp.concatenate(
        [bhh_ref[...], jnp.zeros((3 * _H, 2), jnp.float32)],
        axis=1)

    ones_gc = jnp.ones((1, _G * _C), jnp.float32)
    w_all = jnp.concatenate([w_row] * _G, axis=1)
    blk_iota = jax.lax.broadcasted_iota(
        jnp.int32, (_G, _G * _C), 1) // _C
    sub_iota = jax.lax.broadcasted_iota(
        jnp.int32, (_G, _G * _C), 0)
    blk_sel = (sub_iota == blk_iota).astype(jnp.float32)

    out_groups = []
    for g in range(_BT // _G):
        js = range(g * _G, (g + 1) * _G)
        z_parts, gh_parts = [], []
        for j in js:
            ceq_row = ceq_rows[j]
            mf_row = mf_ref[j:j + 1, :]
            adj0_row = adj0_ref[j:j + 1, :]
            adj1_row = adj1_ref[j:j + 1, :]

            cols_fn = jnp.concatenate(
                [colones_fn[:, j:j + 1], qd, colceq_fn[:, j:j + 1]],
                axis=1)
            w_aug = jnp.concatenate(
                [whtcat_ref[...],
                 jnp.concatenate([cols_fn, cols_gru], axis=0)], axis=1)
            aug_in = jnp.concatenate(
                [ht_ref[j], ones_c, mf_row, ceq_row], axis=0)
            big = jnp.dot(w_aug, aug_in,
                          preferred_element_type=jnp.float32)
            h1 = jnp.maximum(big[:2 * _H], 0.0)
            gh_parts.append(big[2 * _H:])

            z_parts.append(jnp.concatenate(
                [h1[:_H] * adj0_row, h1[_H:] * adj1_row,
                 adj0_row, adj1_row], axis=0))

        z_all = jnp.concatenate(z_parts, axis=1)
        gh_all = jnp.concatenate(gh_parts, axis=1)
        ceq_all = jnp.concatenate(
            [ceq_rows[j] for j in js], axis=1)
        htT_all = jnp.concatenate(
            [ht_ref[j] for j in js], axis=1)

        nf_all = jnp.dot(w2aug_ref[...], z_all,
                         preferred_element_type=jnp.float32)

        t_g = blk_sel * ceq_all
        outer_all = jnp.dot(self_feat_all[:, g * _G:(g + 1) * _G], t_g,
                            preferred_element_type=jnp.float32)
        m_next = nf_all * (1.0 - ceq_all) + outer_all

        ea = jnp.dot(eawaug_ref[...],
                     jnp.concatenate([m_next, ones_gc], axis=0),
                     preferred_element_type=jnp.float32)
        eg = _sigp(ea[:_H])
        ag = jnp.tanh(ea[_H:])
        m2 = m_next - (w_all * eg) * m_next + w_all * ag

        gi = jnp.dot(wihaug_ref[...],
                     jnp.concatenate([m2, ones_gc], axis=0),
                     preferred_element_type=jnp.float32)
        r = _sigp(gi[:_H] + gh_all[:_H])
        zg = _sigp(gi[_H:2 * _H] + gh_all[_H:2 * _H])
        n = jnp.tanh(gi[2 * _H:] + r * gh_all[2 * _H:])
        h_next = n + zg * (htT_all - n)

        out_groups.append(_sigp(jnp.dot(
            wpaug_ref[...],
            jnp.concatenate([h_next, ones_gc], axis=0),
            preferred_element_type=jnp.float32)))

    out_ref[0] = jnp.concatenate(out_groups, axis=1)


def kernel(xt, qt, ht, qt_kc, emb_x_table, emb_c_table, se_w1, se_w2,
           fs_w1, fs_b1, fs_w2, fs_b2, fn_w1, fn_b1, fn_w2, fn_b2,
           ea_w, ea_we, ea_be, ea_wa, ea_ba,
           gru_wih, gru_bih, gru_whh, gru_bhh, wp, bp, graphs):
    f32 = jnp.float32
    x_emb = emb_x_table[:_C]

    wh_T = jnp.concatenate([fn_w1[0, _D:_D + _H].T,
                            fn_w1[1, _D:_D + _H].T], axis=0)
    wq_T = jnp.concatenate([fn_w1[0, _D + _H:].T,
                            fn_w1[1, _D + _H:].T], axis=0)
    wself_T = jnp.concatenate([fn_w1[0, :_D].T,
                               fn_w1[1, :_D].T], axis=0)
    b1cat = jnp.concatenate([fn_b1[0], fn_b1[1]]).reshape(2 * _H, 1)
    w2cat_T = jnp.concatenate([fn_w2[0].T, fn_w2[1].T], axis=1)
    b2_T = jnp.stack([fn_b2[0], fn_b2[1]], axis=1)
    eacat_T = jnp.concatenate([0.5 * ea_we.T, ea_wa.T], axis=0)
    whh_s = jnp.concatenate([0.5 * gru_whh.T[:2 * _H],
                             gru_whh.T[2 * _H:]], axis=0)
    whtcat = jnp.concatenate([wh_T, whh_s], axis=0)
    bhh_s = jnp.concatenate([0.5 * gru_bhh[:2 * _H],
                             gru_bhh[2 * _H:]])
    wih_s = jnp.concatenate([0.5 * gru_wih.T[:2 * _H],
                             gru_wih.T[2 * _H:]], axis=0)
    bih_s = jnp.concatenate([0.5 * gru_bih[:2 * _H],
                             gru_bih[2 * _H:]])

    pa_w = jnp.concatenate([
        jnp.concatenate([fs_w1.T, fs_b1.reshape(_H, 1)], axis=1),
        jnp.concatenate([wself_T, b1cat], axis=1),
        jnp.concatenate([jnp.zeros((2 * _H, _H), f32), wq_T,
                         jnp.zeros((2 * _H, 1), f32)], axis=1),
    ], axis=0)
    fsw2_aug = jnp.concatenate([fs_w2.T, fs_b2.reshape(_H, 1)], axis=1)
    w2aug = jnp.concatenate([w2cat_T, b2_T], axis=1)
    ea_w_aug = jnp.concatenate(
        [eacat_T,
         jnp.concatenate([0.5 * ea_be, ea_ba]).reshape(2 * _H, 1)], axis=1)
    wih_aug = jnp.concatenate([wih_s, bih_s.reshape(3 * _H, 1)],
                              axis=1)
    wp_aug = 0.5 * jnp.concatenate([wp.reshape(1, _H), bp.reshape(1, 1)],
                                   axis=1)

    aux, mf, adj0, adj1, res = pl.pallas_call(
        _prologue_kernel,
        out_shape=(jax.ShapeDtypeStruct((2 * _H, 3), f32),
                   jax.ShapeDtypeStruct((_B, _C), f32),
                   jax.ShapeDtypeStruct((_B, _C), f32),
                   jax.ShapeDtypeStruct((_B, _C), f32),
                   jax.ShapeDtypeStruct((_B, _E), f32)),
    )(x_emb, se_w1, se_w2, emb_c_table, wq_T,
      qt.astype(f32).reshape(_B, 1), xt.reshape(_B, 1),
      qt_kc[:_C], graphs)

    operands = (
        jnp.transpose(ht, (0, 2, 1)), mf, adj0, adj1, res,
        aux, whtcat, bhh_s.reshape(3 * _H, 1),
        pa_w, fsw2_aug, w2aug,
        ea_w_aug, wih_aug, ea_w.reshape(1, _C), wp_aug,
    )

    def full(a):
        nd = a.ndim
        return pl.BlockSpec(a.shape, lambda i, q, _n=nd: (0,) * _n)

    in_specs = [
        pl.BlockSpec((_BT, _H, _C), lambda i, q: (i, 0, 0)),
        pl.BlockSpec((_BT, _C), lambda i, q: (i, 0)),
        pl.BlockSpec((_BT, _C), lambda i, q: (i, 0)),
        pl.BlockSpec((_BT, _C), lambda i, q: (i, 0)),
        pl.BlockSpec((_BT, _E), lambda i, q: (i, 0)),
    ] + [full(a) for a in operands[5:]]

    grid_spec = pltpu.PrefetchScalarGridSpec(
        num_scalar_prefetch=1,
        grid=(_B // _BT,),
        in_specs=in_specs,
        out_specs=pl.BlockSpec((1, 1, _BT * _C), lambda i, q: (i, 0, 0)),
    )

    yt = pl.pallas_call(
        _gkt_kernel,
        grid_spec=grid_spec,
        out_shape=jax.ShapeDtypeStruct((_B // _BT, 1, _BT * _C), f32),
    )(qt, *operands)
    return yt.reshape(_B, _C)
